# baseline (speedup 1.0000x reference)
import functools
import jax
import jax.numpy as jnp
from jax import lax
from jax.experimental import pallas as pl
from jax.experimental.pallas import tpu as pltpu
from jax.experimental.pallas import tpu_sc as plsc

N = 10000
E = 320000
D = 128
DELTA = 3.5
NW = 32
NPT = 320
N_PAD = NW * NPT
EB = 2000
NSTEP = EB // 16
NBLK = E // EB
G = 64
CB = EB + 16
NEG = -3.0e38
POS = 3.0e38


def _mm_kernel(x_ref, w_ref, b_ref, o_ref):
    o_ref[...] = jnp.dot(x_ref[...], w_ref[...],
                         preferred_element_type=jnp.float32) + b_ref[...]


def _matmul(x, w, b, blk):
    M, K = x.shape
    _, F = w.shape
    return pl.pallas_call(
        _mm_kernel,
        grid=(M // blk,),
        in_specs=[pl.BlockSpec((blk, K), lambda i: (i, 0)),
                  pl.BlockSpec((K, F), lambda i: (0, 0)),
                  pl.BlockSpec((1, F), lambda i: (0, 0))],
        out_specs=pl.BlockSpec((blk, F), lambda i: (i, 0)),
        out_shape=jax.ShapeDtypeStruct((M, F), jnp.float32),
    )(x, w, b.reshape(1, F))


def _make_seg_kernel(pass_id):
    mesh = plsc.VectorSubcoreMesh(core_axis_name="c", subcore_axis_name="s")
    if pass_id == 0:
        out_type = [
            jax.ShapeDtypeStruct((N_PAD, D), jnp.float32),
            jax.ShapeDtypeStruct((N_PAD, D), jnp.float32),
            jax.ShapeDtypeStruct((N_PAD * 16,), jnp.float32),
        ]
    else:
        out_type = [
            jax.ShapeDtypeStruct((N_PAD, D), jnp.float32),
            jax.ShapeDtypeStruct((N_PAD, D), jnp.float32),
        ]
    scratch = [
        pltpu.VMEM((EB,), jnp.int32),
        pltpu.VMEM((EB,), jnp.int32),
        pltpu.VMEM((CB,), jnp.int32),
        pltpu.VMEM((CB,), jnp.int32),
        pltpu.VMEM((CB,), jnp.int32),
        pltpu.VMEM((G, D), jnp.float32),
        pltpu.VMEM((G, D), jnp.float32),
        pltpu.VMEM((NPT, D), jnp.float32),
        pltpu.VMEM((NPT, D), jnp.float32),
        pltpu.VMEM((NPT * 16,), jnp.float32),
        pltpu.SemaphoreType.DMA,
        pltpu.SemaphoreType.DMA,
    ]

    @functools.partial(
        pl.kernel, out_type=out_type, mesh=mesh, scratch_types=scratch,
        compiler_params=pltpu.CompilerParams(needs_layout_passes=False))
    def seg(A_h, C_h, src_h, dst_h, *refs):
        outs = refs[:len(out_type)]
        (dst_v, src_v, dl_cbuf, src_cbuf, eid_cbuf, arow, crow,
         acc0, acc1, acc_deg, sem_a, sem_c) = refs[len(out_type):]
        wid = lax.axis_index("s") * 2 + lax.axis_index("c")
        lo = wid * NPT
        zeros = jnp.zeros((16,), jnp.float32)
        ones = jnp.ones((16,), jnp.float32)
        init0 = zeros if pass_id == 0 else jnp.full((16,), NEG, jnp.float32)
        init1 = zeros if pass_id == 0 else jnp.full((16,), POS, jnp.float32)
        zi = jnp.zeros((16,), jnp.int32)
        iota = lax.iota(jnp.int32, 16)

        def init_row(r, carry):
            for c in range(D // 16):
                s = pl.ds(c * 16, 16)
                acc0[r, s] = init0
                acc1[r, s] = init1
            if pass_id == 0:
                acc_deg[pl.ds(r * 16, 16)] = zeros
            return carry
        lax.fori_loop(0, NPT, init_row, 0)

        def init_cb(r, carry):
            s = pl.ds(r * 16, 16)
            dl_cbuf[s] = zi
            src_cbuf[s] = zi
            eid_cbuf[s] = zi
            return carry
        lax.fori_loop(0, CB // 16, init_cb, 0)

        def block_body(b, carry):
            e0 = b * EB
            pltpu.sync_copy(dst_h.at[pl.ds(e0, EB)], dst_v)
            pltpu.sync_copy(src_h.at[pl.ds(e0, EB)], src_v)

            def scan_step(j, cnt):
                s = pl.ds(j * 16, 16)
                dl = dst_v[s] - lo
                mask = (dl >= 0) & (dl < NPT)
                mi = mask.astype(jnp.int32)
                pos = cnt + plsc.cumsum(mi) - 1
                plsc.store_scatter(dl_cbuf, [pos], dl, mask=mask)
                plsc.store_scatter(src_cbuf, [pos], src_v[s], mask=mask)
                plsc.store_scatter(eid_cbuf, [pos], (e0 + j * 16) + iota,
                                   mask=mask)
                return cnt + jnp.sum(mi)

            cnt = lax.fori_loop(0, NSTEP, scan_step, jnp.int32(0))

            nsub = (cnt + (G - 1)) // G

            def sub_body(t, carry2):
                cpa = pltpu.async_copy(A_h.at[src_cbuf.at[pl.ds(t * G, G)]],
                                       arow, sem_a)
                cpc = pltpu.async_copy(C_h.at[eid_cbuf.at[pl.ds(t * G, G)]],
                                       crow, sem_c)
                cpa.wait()
                cpc.wait()
                nedge = jnp.minimum(cnt - t * G, G)

                def edge_body(i, c2):
                    dl2 = dl_cbuf[pl.ds(t * G + i, 16)][0]
                    for c in range(D // 16):
                        s = pl.ds(c * 16, 16)
                        m = arow[i, s] + crow[i, s]
                        if pass_id == 0:
                            acc0[dl2, s] += m
                            acc1[dl2, s] += m * m
                        else:
                            acc0[dl2, s] = jnp.maximum(acc0[dl2, s], m)
                            acc1[dl2, s] = jnp.minimum(acc1[dl2, s], m)
                    if pass_id == 0:
                        acc_deg[pl.ds(dl2 * 16, 16)] += ones
                    return c2
                lax.fori_loop(0, nedge, edge_body, 0)
                return carry2

            lax.fori_loop(0, nsub, sub_body, 0)
            return carry

        lax.fori_loop(0, NBLK, block_body, 0)

        pltpu.sync_copy(acc0, outs[0].at[pl.ds(lo, NPT)])
        pltpu.sync_copy(acc1, outs[1].at[pl.ds(lo, NPT)])
        if pass_id == 0:
            pltpu.sync_copy(acc_deg, outs[2].at[pl.ds(lo * 16, NPT * 16)])

    return seg


_seg_sum = _make_seg_kernel(0)
_seg_ext = _make_seg_kernel(1)


_SCALE = 0.01


def _post_kernel(nf, bb, sm, sq, mx_, mn_, dg, wu, bu, o_hp, o_cs, o_cq):
    i = pl.program_id(0)
    Sm = sm[...]
    Sq = sq[...]
    Mx = mx_[...]
    Mn = mn_[...]
    deg = dg[...][:, 0:1]
    B = bb[...]
    has = deg > 0
    safe = jnp.where(has, deg, 1.0)
    s_full = Sm + deg * B
    ssq_full = Sq + 2.0 * B * Sm + deg * B * B
    mean = s_full / safe
    mean_sq = ssq_full / safe
    var = jnp.maximum(mean_sq - mean * mean, 0.0)
    std = jnp.sqrt(var + 1e-30)
    mx = jnp.where(has, Mx + B, 0.0)
    mn = jnp.where(has, Mn + B, 0.0)
    h = jnp.concatenate([mean, mx, mn, std], axis=1)
    logd = jnp.log(deg + 1.0)
    amp = logd / DELTA
    att = jnp.where(logd > 0, DELTA / jnp.where(logd > 0, logd, 1.0), 0.0)
    hcat = jnp.concatenate([nf[...], h, h * amp, h * att], axis=1)
    hp = (jnp.dot(hcat, wu[...], preferred_element_type=jnp.float32)
          + bu[...]) * _SCALE
    o_hp[...] = hp
    cs = jnp.sum(hp, axis=0, keepdims=True)
    cq = jnp.sum(hp * hp, axis=0, keepdims=True)

    @pl.when(i == 0)
    def _():
        o_cs[...] = cs
        o_cq[...] = cq

    @pl.when(i != 0)
    def _():
        o_cs[...] += cs
        o_cq[...] += cq


def _post(n_feat, B, Sm, Sq, Mx, Mn, dg, W_U, b_U, blk=400):
    row = pl.BlockSpec((blk, D), lambda i: (i, 0))
    return pl.pallas_call(
        _post_kernel,
        grid=(N // blk,),
        in_specs=[row, row, row, row, row, row,
                  pl.BlockSpec((blk, 16), lambda i: (i, 0)),
                  pl.BlockSpec((13 * D, D), lambda i: (0, 0)),
                  pl.BlockSpec((1, D), lambda i: (0, 0))],
        out_specs=[row,
                   pl.BlockSpec((1, D), lambda i: (0, 0)),
                   pl.BlockSpec((1, D), lambda i: (0, 0))],
        out_shape=[jax.ShapeDtypeStruct((N, D), jnp.float32),
                   jax.ShapeDtypeStruct((1, D), jnp.float32),
                   jax.ShapeDtypeStruct((1, D), jnp.float32)],
    )(n_feat, B, Sm, Sq, Mx, Mn, dg, W_U, b_U.reshape(1, D))


def _final_kernel(hp, nf, mu, inv, bt, wm, bm, o):
    h_bn = (hp[...] - mu[...]) * inv[...] + bt[...]
    y = jnp.dot(h_bn, wm[...], preferred_element_type=jnp.float32) + bm[...]
    y = jnp.where(y >= 0, y, 0.01 * y)
    o[...] = jnp.maximum(y + nf[...], 0.0)


def _final(hp, n_feat, mu, inv, beta, W_mix, b_mix, blk=400):
    row = pl.BlockSpec((blk, D), lambda i: (i, 0))
    one = pl.BlockSpec((1, D), lambda i: (0, 0))
    return pl.pallas_call(
        _final_kernel,
        grid=(N // blk,),
        in_specs=[row, row, one, one, one,
                  pl.BlockSpec((D, D), lambda i: (0, 0)), one],
        out_specs=row,
        out_shape=jax.ShapeDtypeStruct((N, D), jnp.float32),
    )(hp, n_feat, mu.reshape(1, D), inv.reshape(1, D), beta.reshape(1, D),
      W_mix, b_mix.reshape(1, D))


def kernel(n_feat, e_feat, W_M, b_M, W_U, b_U, gamma, beta, W_mix, b_mix,
           edge_index):
    src = edge_index[0]
    dst = edge_index[1]

    W_AB = jnp.concatenate([W_M[:D], W_M[D:2 * D]], axis=1)
    b_AB = jnp.concatenate([jnp.zeros_like(b_M), b_M])
    AB = _matmul(n_feat, W_AB, b_AB, 400)
    A, B = AB[:, :D], AB[:, D:]
    C = _matmul(e_feat, W_M[2 * D:], jnp.zeros_like(b_M), 512)

    Sm, Sq, degf = _seg_sum(A, C, src, dst)
    Mx, Mn = _seg_ext(A, C, src, dst)
    dg = degf.reshape(N_PAD, 16)[:N]

    hp, cs, cq = _post(n_feat, B, Sm[:N], Sq[:N], Mx[:N], Mn[:N], dg,
                       W_U, b_U)
    mu = cs[0] / N
    v = jnp.maximum(cq[0] / N - mu * mu, 0.0)
    inv = gamma / jnp.sqrt(v + 1e-5)
    return _final(hp, n_feat, mu, inv, beta, W_mix, b_mix)


# speedup vs baseline: 1.4910x; 1.4910x over previous
import functools
import jax
import jax.numpy as jnp
from jax import lax
from jax.experimental import pallas as pl
from jax.experimental.pallas import tpu as pltpu
from jax.experimental.pallas import tpu_sc as plsc

N = 10000
E = 320000
D = 128
DELTA = 3.5
NW = 32
NPT = 320
N_PAD = NW * NPT
EB = 1280
NSTEP = EB // 16
NBLK = E // EB
G = 64
CB = EB + 16
NEG = -3.0e38
POS = 3.0e38


def _mm_kernel(x_ref, w_ref, b_ref, o_ref):
    o_ref[...] = jnp.dot(x_ref[...], w_ref[...],
                         preferred_element_type=jnp.float32) + b_ref[...]


def _matmul(x, w, b, blk):
    M, K = x.shape
    _, F = w.shape
    return pl.pallas_call(
        _mm_kernel,
        grid=(M // blk,),
        in_specs=[pl.BlockSpec((blk, K), lambda i: (i, 0)),
                  pl.BlockSpec((K, F), lambda i: (0, 0)),
                  pl.BlockSpec((1, F), lambda i: (0, 0))],
        out_specs=pl.BlockSpec((blk, F), lambda i: (i, 0)),
        out_shape=jax.ShapeDtypeStruct((M, F), jnp.float32),
    )(x, w, b.reshape(1, F))


def _make_seg_kernel(pass_id):
    mesh = plsc.VectorSubcoreMesh(core_axis_name="c", subcore_axis_name="s")
    if pass_id == 0:
        out_type = [
            jax.ShapeDtypeStruct((N_PAD, D), jnp.float32),
            jax.ShapeDtypeStruct((N_PAD, D), jnp.float32),
            jax.ShapeDtypeStruct((N_PAD * 16,), jnp.float32),
        ]
    else:
        out_type = [
            jax.ShapeDtypeStruct((N_PAD, D), jnp.float32),
            jax.ShapeDtypeStruct((N_PAD, D), jnp.float32),
        ]
    scratch = [
        pltpu.VMEM((EB,), jnp.int32),
        pltpu.VMEM((EB,), jnp.int32),
        pltpu.VMEM((EB,), jnp.int32),
        pltpu.VMEM((EB,), jnp.int32),
        pltpu.VMEM((CB,), jnp.int32),
        pltpu.VMEM((CB,), jnp.int32),
        pltpu.VMEM((CB,), jnp.int32),
        pltpu.VMEM((2, G), jnp.int32),
        pltpu.VMEM((2, G), jnp.int32),
        pltpu.VMEM((2, 128), jnp.int32),
        pltpu.VMEM((G, D), jnp.float32),
        pltpu.VMEM((G, D), jnp.float32),
        pltpu.VMEM((G, D), jnp.float32),
        pltpu.VMEM((G, D), jnp.float32),
        pltpu.VMEM((NPT, D), jnp.float32),
        pltpu.VMEM((NPT, D), jnp.float32),
        pltpu.VMEM((NPT * 16,), jnp.float32),
        pltpu.SemaphoreType.DMA,
        pltpu.SemaphoreType.DMA,
        pltpu.SemaphoreType.DMA,
        pltpu.SemaphoreType.DMA,
        pltpu.SemaphoreType.DMA,
        pltpu.SemaphoreType.DMA,
        pltpu.SemaphoreType.DMA,
        pltpu.SemaphoreType.DMA,
    ]

    @functools.partial(
        pl.kernel, out_type=out_type, mesh=mesh, scratch_types=scratch,
        compiler_params=pltpu.CompilerParams(needs_layout_passes=False))
    def seg(A_h, C_h, src_h, dst_h, *refs):
        outs = refs[:len(out_type)]
        (dst0, dst1, src0, src1, dl_cb, src_cb, eid_cb,
         sv_src, sv_eid, sv_dl, ga0, ga1, gc0, gc1,
         acc0, acc1, acc_deg,
         st_d0, st_d1, st_s0, st_s1, g_a0, g_a1, g_c0, g_c1
         ) = refs[len(out_type):]
        dstb = (dst0, dst1)
        srcb = (src0, src1)
        gab = (ga0, ga1)
        gcb = (gc0, gc1)
        st_d = (st_d0, st_d1)
        st_s = (st_s0, st_s1)
        g_a = (g_a0, g_a1)
        g_c = (g_c0, g_c1)

        wid = lax.axis_index("s") * 2 + lax.axis_index("c")
        lo = wid * NPT
        zeros = jnp.zeros((16,), jnp.float32)
        ones = jnp.ones((16,), jnp.float32)
        init0 = zeros if pass_id == 0 else jnp.full((16,), NEG, jnp.float32)
        init1 = zeros if pass_id == 0 else jnp.full((16,), POS, jnp.float32)
        zi = jnp.zeros((16,), jnp.int32)
        iota = lax.iota(jnp.int32, 16)

        def init_row(r, carry):
            for c in range(D // 16):
                s = pl.ds(c * 16, 16)
                acc0[r, s] = init0
                acc1[r, s] = init1
            if pass_id == 0:
                acc_deg[pl.ds(r * 16, 16)] = zeros
            return carry
        lax.fori_loop(0, NPT, init_row, 0)

        def init_cb(r, carry):
            s = pl.ds(r * 16, 16)
            dl_cb[s] = zi
            src_cb[s] = zi
            eid_cb[s] = zi
            return carry
        lax.fori_loop(0, CB // 16, init_cb, 0)
        for sl in range(2):
            for k in range(G // 16):
                s = pl.ds(k * 16, 16)
                sv_src[sl, s] = zi
                sv_eid[sl, s] = zi
            for k in range(8):
                sv_dl[sl, pl.ds(k * 16, 16)] = zi

        def stage_issue(sl, b):
            e0 = jnp.minimum(b, NBLK - 1) * EB
            pltpu.make_async_copy(dst_h.at[pl.ds(e0, EB)], dstb[sl],
                                  st_d[sl]).start()
            pltpu.make_async_copy(src_h.at[pl.ds(e0, EB)], srcb[sl],
                                  st_s[sl]).start()

        def stage_wait(sl, b):
            e0 = jnp.minimum(b, NBLK - 1) * EB
            pltpu.make_async_copy(dst_h.at[pl.ds(e0, EB)], dstb[sl],
                                  st_d[sl]).wait()
            pltpu.make_async_copy(src_h.at[pl.ds(e0, EB)], srcb[sl],
                                  st_s[sl]).wait()

        def gather_issue(sl):
            pltpu.make_async_copy(A_h.at[sv_src.at[sl]], gab[sl],
                                  g_a[sl]).start()
            pltpu.make_async_copy(C_h.at[sv_eid.at[sl]], gcb[sl],
                                  g_c[sl]).start()

        def gather_wait(sl):
            pltpu.make_async_copy(A_h.at[sv_src.at[sl]], gab[sl],
                                  g_a[sl]).wait()
            pltpu.make_async_copy(C_h.at[sv_eid.at[sl]], gcb[sl],
                                  g_c[sl]).wait()

        def accum_edge(dl2, arow, crow, i):
            for c in range(D // 16):
                s = pl.ds(c * 16, 16)
                m = arow[i, s] + crow[i, s]
                if pass_id == 0:
                    acc0[dl2, s] += m
                    acc1[dl2, s] += m * m
                else:
                    acc0[dl2, s] = jnp.maximum(acc0[dl2, s], m)
                    acc1[dl2, s] = jnp.minimum(acc1[dl2, s], m)
            if pass_id == 0:
                acc_deg[pl.ds(dl2 * 16, 16)] += ones

        def phase(b, sl, cnt_prev):
            stage_wait(sl, b)
            dv = dstb[sl]
            sv = srcb[sl]

            def scan_step(j, cnt):
                s = pl.ds(j * 16, 16)
                dl = dv[s] - lo
                mask = (dl >= 0) & (dl < NPT)
                mi = mask.astype(jnp.int32)
                P = plsc.cumsum(mi)
                pos = cnt + P - 1
                plsc.store_scatter(dl_cb, [pos], dl, mask=mask)
                plsc.store_scatter(src_cb, [pos], sv[s], mask=mask)
                plsc.store_scatter(eid_cb, [pos], (b * EB + j * 16) + iota,
                                   mask=mask)
                return cnt + P[15]

            cnt = lax.fori_loop(0, NSTEP, scan_step, jnp.int32(0))

            nov = lax.max(cnt - G + 15, 0) // 16

            def ov_body(t, carry2):
                base = G + t * 16
                ov_a = gab[sl].at[pl.ds(0, 16)]
                ov_c = gcb[sl].at[pl.ds(0, 16)]
                pltpu.make_async_copy(A_h.at[src_cb.at[pl.ds(base, 16)]],
                                      ov_a, g_a[sl]).start()
                pltpu.make_async_copy(C_h.at[eid_cb.at[pl.ds(base, 16)]],
                                      ov_c, g_c[sl]).start()
                pltpu.make_async_copy(A_h.at[src_cb.at[pl.ds(base, 16)]],
                                      ov_a, g_a[sl]).wait()
                pltpu.make_async_copy(C_h.at[eid_cb.at[pl.ds(base, 16)]],
                                      ov_c, g_c[sl]).wait()
                kmax = jnp.minimum(cnt - base, 16)

                def ov_edge(k, c3):
                    dl2 = dl_cb[pl.ds(base + k, 16)][0]
                    accum_edge(dl2, gab[sl], gcb[sl], k)
                    return c3
                lax.fori_loop(0, kmax, ov_edge, 0)
                return carry2
            lax.fori_loop(0, nov, ov_body, 0)

            for k in range(G // 16):
                s = pl.ds(k * 16, 16)
                sv_src[sl, s] = src_cb[s]
                sv_eid[sl, s] = eid_cb[s]
                sv_dl[sl, s] = dl_cb[s]
            gather_issue(sl)
            stage_issue(sl, b + 2)

            osl = 1 - sl
            gather_wait(osl)

            def edge_body(i, c2):
                dl2 = sv_dl[osl, pl.ds(i, 16)][0]
                accum_edge(dl2, gab[osl], gcb[osl], i)
                return c2
            lax.fori_loop(0, cnt_prev, edge_body, 0)
            return jnp.minimum(cnt, G)

        stage_issue(0, 0)
        stage_issue(1, 1)
        gather_issue(1)

        def pair_body(i, cnt_prev):
            c0 = phase(2 * i, 0, cnt_prev)
            c1 = phase(2 * i + 1, 1, c0)
            return c1
        cnt_last = lax.fori_loop(0, NBLK // 2, pair_body, jnp.int32(0))

        gather_wait(1)

        def last_edges(i, c2):
            dl2 = sv_dl[1, pl.ds(i, 16)][0]
            accum_edge(dl2, gab[1], gcb[1], i)
            return c2
        lax.fori_loop(0, cnt_last, last_edges, 0)
        stage_wait(0, NBLK)
        stage_wait(1, NBLK + 1)

        pltpu.sync_copy(acc0, outs[0].at[pl.ds(lo, NPT)])
        pltpu.sync_copy(acc1, outs[1].at[pl.ds(lo, NPT)])
        if pass_id == 0:
            pltpu.sync_copy(acc_deg, outs[2].at[pl.ds(lo * 16, NPT * 16)])

    return seg


_seg_sum = _make_seg_kernel(0)
_seg_ext = _make_seg_kernel(1)


_SCALE = 0.01


def _post_kernel(nf, bb, sm, sq, mx_, mn_, dg, wu, bu, o_hp, o_cs, o_cq):
    i = pl.program_id(0)
    Sm = sm[...]
    Sq = sq[...]
    Mx = mx_[...]
    Mn = mn_[...]
    deg = dg[...][:, 0:1]
    B = bb[...]
    has = deg > 0
    safe = jnp.where(has, deg, 1.0)
    s_full = Sm + deg * B
    ssq_full = Sq + 2.0 * B * Sm + deg * B * B
    mean = s_full / safe
    mean_sq = ssq_full / safe
    var = jnp.maximum(mean_sq - mean * mean, 0.0)
    std = jnp.sqrt(var + 1e-30)
    mx = jnp.where(has, Mx + B, 0.0)
    mn = jnp.where(has, Mn + B, 0.0)
    h = jnp.concatenate([mean, mx, mn, std], axis=1)
    logd = jnp.log(deg + 1.0)
    amp = logd / DELTA
    att = jnp.where(logd > 0, DELTA / jnp.where(logd > 0, logd, 1.0), 0.0)
    hcat = jnp.concatenate([nf[...], h, h * amp, h * att], axis=1)
    hp = (jnp.dot(hcat, wu[...], preferred_element_type=jnp.float32)
          + bu[...]) * _SCALE
    o_hp[...] = hp
    cs = jnp.sum(hp, axis=0, keepdims=True)
    cq = jnp.sum(hp * hp, axis=0, keepdims=True)

    @pl.when(i == 0)
    def _():
        o_cs[...] = cs
        o_cq[...] = cq

    @pl.when(i != 0)
    def _():
        o_cs[...] += cs
        o_cq[...] += cq


def _post(n_feat, B, Sm, Sq, Mx, Mn, dg, W_U, b_U, blk=400):
    row = pl.BlockSpec((blk, D), lambda i: (i, 0))
    return pl.pallas_call(
        _post_kernel,
        grid=(N // blk,),
        in_specs=[row, row, row, row, row, row,
                  pl.BlockSpec((blk, 16), lambda i: (i, 0)),
                  pl.BlockSpec((13 * D, D), lambda i: (0, 0)),
                  pl.BlockSpec((1, D), lambda i: (0, 0))],
        out_specs=[row,
                   pl.BlockSpec((1, D), lambda i: (0, 0)),
                   pl.BlockSpec((1, D), lambda i: (0, 0))],
        out_shape=[jax.ShapeDtypeStruct((N, D), jnp.float32),
                   jax.ShapeDtypeStruct((1, D), jnp.float32),
                   jax.ShapeDtypeStruct((1, D), jnp.float32)],
    )(n_feat, B, Sm, Sq, Mx, Mn, dg, W_U, b_U.reshape(1, D))


def _final_kernel(hp, nf, mu, inv, bt, wm, bm, o):
    h_bn = (hp[...] - mu[...]) * inv[...] + bt[...]
    y = jnp.dot(h_bn, wm[...], preferred_element_type=jnp.float32) + bm[...]
    y = jnp.where(y >= 0, y, 0.01 * y)
    o[...] = jnp.maximum(y + nf[...], 0.0)


def _final(hp, n_feat, mu, inv, beta, W_mix, b_mix, blk=400):
    row = pl.BlockSpec((blk, D), lambda i: (i, 0))
    one = pl.BlockSpec((1, D), lambda i: (0, 0))
    return pl.pallas_call(
        _final_kernel,
        grid=(N // blk,),
        in_specs=[row, row, one, one, one,
                  pl.BlockSpec((D, D), lambda i: (0, 0)), one],
        out_specs=row,
        out_shape=jax.ShapeDtypeStruct((N, D), jnp.float32),
    )(hp, n_feat, mu.reshape(1, D), inv.reshape(1, D), beta.reshape(1, D),
      W_mix, b_mix.reshape(1, D))


def kernel(n_feat, e_feat, W_M, b_M, W_U, b_U, gamma, beta, W_mix, b_mix,
           edge_index):
    src = edge_index[0]
    dst = edge_index[1]

    A = _matmul(n_feat, W_M[:D], jnp.zeros_like(b_M), 400)
    B = _matmul(n_feat, W_M[D:2 * D], b_M, 400)
    C = _matmul(e_feat, W_M[2 * D:], jnp.zeros_like(b_M), 512)

    Sm, Sq, degf = _seg_sum(A, C, src, dst)
    Mx, Mn = _seg_ext(A, C, src, dst)
    dg = degf.reshape(N_PAD, 16)[:N]

    hp, cs, cq = _post(n_feat, B, Sm[:N], Sq[:N], Mx[:N], Mn[:N], dg,
                       W_U, b_U)
    mu = cs[0] / N
    v = jnp.maximum(cq[0] / N - mu * mu, 0.0)
    inv = gamma / jnp.sqrt(v + 1e-5)
    return _final(hp, n_feat, mu, inv, beta, W_mix, b_mix)
